# baseline (speedup 1.0000x reference)
import functools

import jax
import jax.numpy as jnp
from jax import lax
from jax.experimental import pallas as pl
from jax.experimental.pallas import tpu as pltpu
from jax.experimental.pallas import tpu_sc as plsc

DIM = 5
DPAD = 16
NC, NS = 2, 16
NW = NC * NS

BM = 128


def _make_gather(B):
    b_per_w = B // NW
    mesh = plsc.VectorSubcoreMesh(core_axis_name="c", subcore_axis_name="s")

    @functools.partial(
        pl.kernel,
        mesh=mesh,
        out_type=jax.ShapeDtypeStruct((B, DPAD), jnp.float32),
        scratch_types=[
            pltpu.VMEM((b_per_w,), jnp.int32),
            pltpu.VMEM((b_per_w, DPAD), jnp.float32),
            pltpu.SemaphoreType.DMA,
        ],
        compiler_params=pltpu.CompilerParams(use_tc_tiling_on_sc=False),
    )
    def gather(table_hbm, idx_hbm, out_hbm, idx_v, rows_v, sem):
        wid = lax.axis_index("s") * NC + lax.axis_index("c")
        base = wid * b_per_w
        pltpu.sync_copy(idx_hbm.at[pl.ds(base, b_per_w)], idx_v)
        pltpu.async_copy(table_hbm.at[idx_v], rows_v, sem).wait()
        pltpu.sync_copy(rows_v, out_hbm.at[pl.ds(base, b_per_w)])

    return gather


def _proj_body(e_ref, wt_ref, b_ref, o_ref):
    o_ref[...] = (
        jnp.dot(e_ref[...], wt_ref[...], preferred_element_type=jnp.float32)
        + b_ref[...]
    )


def _project(e, wt, bias2d, B, V):
    nb = B // BM
    return pl.pallas_call(
        _proj_body,
        grid=(nb,),
        in_specs=[
            pl.BlockSpec((BM, DPAD), lambda i: (i, 0)),
            pl.BlockSpec((DPAD, V), lambda i: (0, 0)),
            pl.BlockSpec((1, V), lambda i: (0, 0)),
        ],
        out_specs=pl.BlockSpec((BM, V), lambda i: (i, 0)),
        out_shape=jax.ShapeDtypeStruct((B, V), jnp.float32),
    )(e, wt, bias2d)


@jax.jit
def kernel(x, embed_table, fc_weight, fc_bias):
    B = x.shape[0]
    V, dim = embed_table.shape
    table_p = jnp.pad(embed_table, ((0, 0), (0, DPAD - dim)))
    e = _make_gather(B)(table_p, x.astype(jnp.int32))
    wt = jnp.pad(fc_weight, ((0, 0), (0, DPAD - dim))).T
    return _project(e, wt, fc_bias.reshape(1, V), B, V)


# speedup vs baseline: 2.9431x; 2.9431x over previous
import functools

import jax
import jax.numpy as jnp
from jax import lax
from jax.experimental import pallas as pl
from jax.experimental.pallas import tpu as pltpu
from jax.experimental.pallas import tpu_sc as plsc

NC, NS = 2, 16
NW = NC * NS

BMV = 512


def _make_gather(B, V, D):
    b_per_w = B // NW
    mesh = plsc.VectorSubcoreMesh(core_axis_name="c", subcore_axis_name="s")

    @functools.partial(
        pl.kernel,
        mesh=mesh,
        out_type=jax.ShapeDtypeStruct((B, D), jnp.float32),
        scratch_types=[
            pltpu.VMEM((b_per_w,), jnp.int32),
            pltpu.VMEM((b_per_w, D), jnp.float32),
            pltpu.SemaphoreType.DMA,
        ],
        compiler_params=pltpu.CompilerParams(use_tc_tiling_on_sc=False),
    )
    def gather(table_hbm, idx_hbm, out_hbm, idx_v, rows_v, sem):
        wid = lax.axis_index("s") * NC + lax.axis_index("c")
        base = wid * b_per_w
        pltpu.sync_copy(idx_hbm.at[pl.ds(base, b_per_w)], idx_v)
        pltpu.async_copy(table_hbm.at[idx_v], rows_v, sem).wait()
        pltpu.sync_copy(rows_v, out_hbm.at[pl.ds(base, b_per_w)])

    return gather


def _proj_body(w_ref, b_ref, e_ref, o_ref):
    prod = lax.dot_general(
        w_ref[...], e_ref[...],
        dimension_numbers=(((1,), (1,)), ((), ())),
        preferred_element_type=jnp.float32,
    )
    o_ref[...] = prod + b_ref[...][:, None]


def _project_t(w, bias, e, B, V, D):
    nv = pl.cdiv(V, BMV)
    return pl.pallas_call(
        _proj_body,
        grid=(nv,),
        in_specs=[
            pl.BlockSpec((BMV, D), lambda i: (i, 0)),
            pl.BlockSpec((BMV,), lambda i: (i,)),
            pl.BlockSpec((B, D), lambda i: (0, 0)),
        ],
        out_specs=pl.BlockSpec((BMV, B), lambda i: (i, 0)),
        out_shape=jax.ShapeDtypeStruct((V, B), jnp.float32),
    )(w, bias, e)


@jax.jit
def kernel(x, embed_table, fc_weight, fc_bias):
    B = x.shape[0]
    V, D = embed_table.shape
    e = _make_gather(B, V, D)(embed_table, x.astype(jnp.int32))
    out_t = _project_t(fc_weight, fc_bias, e, B, V, D)
    return out_t.T


# speedup vs baseline: 3.6547x; 1.2418x over previous
import functools

import jax
import jax.numpy as jnp
from jax import lax
from jax.experimental import pallas as pl
from jax.experimental.pallas import tpu as pltpu
from jax.experimental.pallas import tpu_sc as plsc

NC, NS = 2, 16
NW = NC * NS
LANES = 16

BMV = 512


def _make_gather_t(B, V, D):
    b_per_w = B // NW
    mesh = plsc.VectorSubcoreMesh(core_axis_name="c", subcore_axis_name="s")

    @functools.partial(
        pl.kernel,
        mesh=mesh,
        out_type=jax.ShapeDtypeStruct((D, B), jnp.float32),
        scratch_types=[
            pltpu.VMEM((b_per_w,), jnp.int32),
            pltpu.VMEM((D, b_per_w), jnp.int32),
            pltpu.VMEM((D, b_per_w), jnp.float32),
            pltpu.SemaphoreType.DMA,
        ],
        compiler_params=pltpu.CompilerParams(use_tc_tiling_on_sc=False),
    )
    def gather(tflat_hbm, idx_hbm, out_hbm, idx_v, idxd_v, rows_v, sem):
        wid = lax.axis_index("s") * NC + lax.axis_index("c")
        base = wid * b_per_w
        pltpu.sync_copy(idx_hbm.at[pl.ds(base, b_per_w)], idx_v)
        for j in range(b_per_w // LANES):
            xv = idx_v[pl.ds(j * LANES, LANES)]
            for d in range(D):
                idxd_v[d, pl.ds(j * LANES, LANES)] = xv + d * V
        copies = [
            pltpu.async_copy(tflat_hbm.at[idxd_v.at[d]], rows_v.at[d], sem)
            for d in range(D)
        ]
        for c in copies:
            c.wait()
        for d in range(D):
            pltpu.sync_copy(rows_v.at[d], out_hbm.at[d, pl.ds(base, b_per_w)])

    return gather


def _proj_body(wt_ref, b_ref, et_ref, o_ref):
    prod = lax.dot_general(
        wt_ref[...], et_ref[...],
        dimension_numbers=(((0,), (0,)), ((), ())),
        preferred_element_type=jnp.float32,
    )
    o_ref[...] = prod + b_ref[...][:, None]


def _project_t(wt, bias, et, B, V, D):
    nv = pl.cdiv(V, BMV)
    return pl.pallas_call(
        _proj_body,
        grid=(nv,),
        in_specs=[
            pl.BlockSpec((D, BMV), lambda i: (0, i)),
            pl.BlockSpec((BMV,), lambda i: (i,)),
            pl.BlockSpec((D, B), lambda i: (0, 0)),
        ],
        out_specs=pl.BlockSpec((BMV, B), lambda i: (i, 0)),
        out_shape=jax.ShapeDtypeStruct((V, B), jnp.float32),
    )(wt, bias, et)


@jax.jit
def kernel(x, embed_table, fc_weight, fc_bias):
    B = x.shape[0]
    V, D = embed_table.shape
    tflat = embed_table.T.reshape(D * V)
    et = _make_gather_t(B, V, D)(tflat, x.astype(jnp.int32))
    wt = fc_weight.T
    out_t = _project_t(wt, fc_bias, et, B, V, D)
    return out_t.T
